# baseline (speedup 1.0000x reference)
import functools

import jax
import jax.numpy as jnp
from jax.experimental import pallas as pl
from jax.experimental.pallas import tpu as pltpu

HID = 20
HPAD = 128
LANES = 65536


def _me_kernel(t_ref, w1_ref, b1_ref, wh_ref, bh_ref, w4_ref, b4_ref, me_ref):
    h = jnp.tanh(t_ref[...] * w1_ref[...] + b1_ref[...])
    for i in range(2):
        h = jnp.tanh(jnp.dot(h, wh_ref[i],
                             preferred_element_type=jnp.float32) + bh_ref[i])
    z = jnp.dot(h, w4_ref[...], preferred_element_type=jnp.float32) + b4_ref[...]
    me_ref[...] = 0.5 * (jnp.tanh(0.5 * z) + 1.0)


def _sir_kernel(sc_ref,
                y0_ref,
                sol_ref, diff_ref,
                *, T):
    beta = sc_ref[0]
    gamma = sc_ref[1]
    dt = sc_ref[2]
    S = y0_ref[0]
    I = y0_ref[1]
    R = y0_ref[2]
    zero = jnp.zeros_like(S)
    I_hist = []
    for u in range(T):
        sol_ref[0, u:u + 1, :] = S
        sol_ref[1, u:u + 1, :] = I
        sol_ref[2, u:u + 1, :] = R
        if u == T - 1:
            diff_ref[0, u:u + 1, :] = zero
            diff_ref[1, u:u + 1, :] = zero
            diff_ref[2, u:u + 1, :] = zero
            break
        I_hist.append(I)
        acc = I_hist[0] * sc_ref[3 + T - 1 - u]
        for k in range(1, u + 1):
            acc = acc + I_hist[k] * sc_ref[3 + T - 1 - u + k]
        SI = S * I
        gI = gamma * I
        dSdt = acc - beta * SI
        dIdt = beta * SI - gI
        dRdt = gI - acc
        diff_ref[0, u:u + 1, :] = dSdt
        diff_ref[1, u:u + 1, :] = dIdt
        diff_ref[2, u:u + 1, :] = dRdt
        S = S + dt * dSdt
        I = I + dt * dIdt
        R = R + dt * dRdt


@jax.jit
def kernel(t, y, beta, gamma, w1, b1, w2, b2, w3, b3, w4, b4):
    T = t.shape[0]
    B = y.shape[0]
    f32 = jnp.float32
    t32 = t.astype(f32)
    dt = t32[0, 0] - t32[1, 0]

    w1p = jnp.pad(w1, ((0, 0), (0, HPAD - HID)))
    b1p = jnp.pad(b1, ((0, 0), (0, HPAD - HID)))
    whp = jnp.stack([jnp.pad(w2, ((0, HPAD - HID), (0, HPAD - HID))),
                     jnp.pad(w3, ((0, HPAD - HID), (0, HPAD - HID)))])
    bhp = jnp.stack([jnp.pad(b2, ((0, 0), (0, HPAD - HID))),
                     jnp.pad(b3, ((0, 0), (0, HPAD - HID)))])
    w4p = jnp.pad(w4, ((0, HPAD - HID), (0, 0)))
    b4p = b4.reshape(1, 1)

    vmem = pl.BlockSpec(memory_space=pltpu.MemorySpace.VMEM)
    me = pl.pallas_call(
        _me_kernel,
        out_shape=jax.ShapeDtypeStruct((T, 1), f32),
        in_specs=[vmem] * 7,
        out_specs=vmem,
    )(t32, w1p, b1p, whp, bhp, w4p, b4p)

    sc = jnp.concatenate([jnp.stack([beta.astype(f32), gamma.astype(f32), dt]),
                          dt * me[:, 0]])

    Bp = 128 * pl.cdiv(B, 128)
    y0 = jnp.transpose(y.astype(f32), (2, 1, 0))
    if Bp != B:
        y0 = jnp.pad(y0, ((0, 0), (0, 0), (0, Bp - B)))

    n_tiles = pl.cdiv(Bp, LANES)
    blk_bytes = 4 * LANES * (3 + 6 * T)
    sol3, dif3 = pl.pallas_call(
        functools.partial(_sir_kernel, T=T),
        out_shape=[jax.ShapeDtypeStruct((3, T, Bp), f32)] * 2,
        grid=(n_tiles,),
        in_specs=[
            pl.BlockSpec(memory_space=pltpu.MemorySpace.SMEM),
            pl.BlockSpec((3, 1, LANES), lambda i: (0, 0, i)),
        ],
        out_specs=[pl.BlockSpec((3, T, LANES), lambda i: (0, 0, i))] * 2,
        compiler_params=pltpu.CompilerParams(
            dimension_semantics=("parallel",),
            vmem_limit_bytes=int(min(max(3 * blk_bytes, 16 << 20), 60 << 20))),
    )(sc, y0)

    solution = jnp.transpose(sol3[:, :, :B], (1, 2, 0))
    diff = jnp.transpose(dif3[:, :, :B], (1, 2, 0))
    return solution, diff, me


# speedup vs baseline: 1.0407x; 1.0407x over previous
import functools

import jax
import jax.numpy as jnp
from jax.experimental import pallas as pl
from jax.experimental.pallas import tpu as pltpu

HID = 20
HPAD = 128
LANES = 32000


def _me_kernel(t_ref, w1_ref, b1_ref, wh_ref, bh_ref, w4_ref, b4_ref, me_ref):
    h = jnp.tanh(t_ref[...] * w1_ref[...] + b1_ref[...])
    for i in range(2):
        h = jnp.tanh(jnp.dot(h, wh_ref[i],
                             preferred_element_type=jnp.float32) + bh_ref[i])
    z = jnp.dot(h, w4_ref[...], preferred_element_type=jnp.float32) + b4_ref[...]
    me_ref[...] = 0.5 * (jnp.tanh(0.5 * z) + 1.0)


def _sir_kernel(sc_ref,
                y0_ref,
                sol_ref, diff_ref,
                *, T):
    beta = sc_ref[0]
    gamma = sc_ref[1]
    dt = sc_ref[2]
    S = y0_ref[0]
    I = y0_ref[1]
    R = y0_ref[2]
    zero = jnp.zeros_like(S)
    I_hist = []
    for u in range(T):
        sol_ref[0, u:u + 1, :] = S
        sol_ref[1, u:u + 1, :] = I
        sol_ref[2, u:u + 1, :] = R
        if u == T - 1:
            diff_ref[0, u:u + 1, :] = zero
            diff_ref[1, u:u + 1, :] = zero
            diff_ref[2, u:u + 1, :] = zero
            break
        I_hist.append(I)
        acc = I_hist[0] * sc_ref[3 + T - 1 - u]
        for k in range(1, u + 1):
            acc = acc + I_hist[k] * sc_ref[3 + T - 1 - u + k]
        SI = S * I
        gI = gamma * I
        dSdt = acc - beta * SI
        dIdt = beta * SI - gI
        dRdt = gI - acc
        diff_ref[0, u:u + 1, :] = dSdt
        diff_ref[1, u:u + 1, :] = dIdt
        diff_ref[2, u:u + 1, :] = dRdt
        S = S + dt * dSdt
        I = I + dt * dIdt
        R = R + dt * dRdt


@jax.jit
def kernel(t, y, beta, gamma, w1, b1, w2, b2, w3, b3, w4, b4):
    T = t.shape[0]
    B = y.shape[0]
    f32 = jnp.float32
    t32 = t.astype(f32)
    dt = t32[0, 0] - t32[1, 0]

    w1p = jnp.pad(w1, ((0, 0), (0, HPAD - HID)))
    b1p = jnp.pad(b1, ((0, 0), (0, HPAD - HID)))
    whp = jnp.stack([jnp.pad(w2, ((0, HPAD - HID), (0, HPAD - HID))),
                     jnp.pad(w3, ((0, HPAD - HID), (0, HPAD - HID)))])
    bhp = jnp.stack([jnp.pad(b2, ((0, 0), (0, HPAD - HID))),
                     jnp.pad(b3, ((0, 0), (0, HPAD - HID)))])
    w4p = jnp.pad(w4, ((0, HPAD - HID), (0, 0)))
    b4p = b4.reshape(1, 1)

    vmem = pl.BlockSpec(memory_space=pltpu.MemorySpace.VMEM)
    me = pl.pallas_call(
        _me_kernel,
        out_shape=jax.ShapeDtypeStruct((T, 1), f32),
        in_specs=[vmem] * 7,
        out_specs=vmem,
    )(t32, w1p, b1p, whp, bhp, w4p, b4p)

    sc = jnp.concatenate([jnp.stack([beta.astype(f32), gamma.astype(f32), dt]),
                          dt * me[:, 0]])

    Bp = 128 * pl.cdiv(B, 128)
    y0 = jnp.transpose(y.astype(f32), (2, 1, 0))
    if Bp != B:
        y0 = jnp.pad(y0, ((0, 0), (0, 0), (0, Bp - B)))

    n_tiles = pl.cdiv(Bp, LANES)
    blk_bytes = 4 * LANES * (3 + 6 * T)
    sol3, dif3 = pl.pallas_call(
        functools.partial(_sir_kernel, T=T),
        out_shape=[jax.ShapeDtypeStruct((3, T, Bp), f32)] * 2,
        grid=(n_tiles,),
        in_specs=[
            pl.BlockSpec(memory_space=pltpu.MemorySpace.SMEM),
            pl.BlockSpec((3, 1, LANES), lambda i: (0, 0, i)),
        ],
        out_specs=[pl.BlockSpec((3, T, LANES), lambda i: (0, 0, i))] * 2,
        compiler_params=pltpu.CompilerParams(
            dimension_semantics=("parallel",),
            vmem_limit_bytes=int(min(max(3 * blk_bytes, 16 << 20), 60 << 20))),
    )(sc, y0)

    solution = jnp.transpose(sol3[:, :, :B], (1, 2, 0))
    diff = jnp.transpose(dif3[:, :, :B], (1, 2, 0))
    return solution, diff, me


# speedup vs baseline: 1.0446x; 1.0037x over previous
import functools

import jax
import jax.numpy as jnp
from jax.experimental import pallas as pl
from jax.experimental.pallas import tpu as pltpu

HID = 20
HPAD = 128
LANES = 32768


def _me_kernel(t_ref, w1_ref, b1_ref, wh_ref, bh_ref, w4_ref, b4_ref, me_ref):
    h = jnp.tanh(w1_ref[...] * t_ref[...] + b1_ref[...])
    for i in range(2):
        h = jnp.tanh(jnp.dot(wh_ref[i], h,
                             preferred_element_type=jnp.float32) + bh_ref[i])
    z = jnp.dot(w4_ref[...], h, preferred_element_type=jnp.float32) + b4_ref[...]
    me_ref[...] = 0.5 * (jnp.tanh(0.5 * z) + 1.0)


def _sir_kernel(sc_ref,
                y0_ref,
                sol_ref, diff_ref,
                *, T):
    beta = sc_ref[0]
    gamma = sc_ref[1]
    dt = sc_ref[2]
    S = y0_ref[0]
    I = y0_ref[1]
    R = y0_ref[2]
    zero = jnp.zeros_like(S)
    I_hist = []
    for u in range(T):
        sol_ref[0, u:u + 1, :] = S
        sol_ref[1, u:u + 1, :] = I
        sol_ref[2, u:u + 1, :] = R
        if u == T - 1:
            diff_ref[0, u:u + 1, :] = zero
            diff_ref[1, u:u + 1, :] = zero
            diff_ref[2, u:u + 1, :] = zero
            break
        I_hist.append(I)
        acc = I_hist[0] * sc_ref[3 + T - 1 - u]
        for k in range(1, u + 1):
            acc = acc + I_hist[k] * sc_ref[3 + T - 1 - u + k]
        SI = S * I
        gI = gamma * I
        dSdt = acc - beta * SI
        dIdt = beta * SI - gI
        dRdt = gI - acc
        diff_ref[0, u:u + 1, :] = dSdt
        diff_ref[1, u:u + 1, :] = dIdt
        diff_ref[2, u:u + 1, :] = dRdt
        S = S + dt * dSdt
        I = I + dt * dIdt
        R = R + dt * dRdt


@jax.jit
def kernel(t, y, beta, gamma, w1, b1, w2, b2, w3, b3, w4, b4):
    T = t.shape[0]
    B = y.shape[0]
    f32 = jnp.float32
    t32 = t.astype(f32)
    dt = t32[0, 0] - t32[1, 0]

    w1p = jnp.pad(w1.T, ((0, HPAD - HID), (0, 0)))
    b1p = jnp.pad(b1.T, ((0, HPAD - HID), (0, 0)))
    whp = jnp.stack([jnp.pad(w2.T, ((0, HPAD - HID), (0, HPAD - HID))),
                     jnp.pad(w3.T, ((0, HPAD - HID), (0, HPAD - HID)))])
    bhp = jnp.stack([jnp.pad(b2.T, ((0, HPAD - HID), (0, 0))),
                     jnp.pad(b3.T, ((0, HPAD - HID), (0, 0)))])
    w4p = jnp.pad(w4.T, ((0, 0), (0, HPAD - HID)))
    b4p = b4.reshape(1, 1)

    vmem = pl.BlockSpec(memory_space=pltpu.MemorySpace.VMEM)
    me_row = pl.pallas_call(
        _me_kernel,
        out_shape=jax.ShapeDtypeStruct((1, T), f32),
        in_specs=[vmem] * 7,
        out_specs=vmem,
    )(t32.reshape(1, T), w1p, b1p, whp, bhp, w4p, b4p)
    me = me_row.T

    sc = jnp.concatenate([jnp.stack([beta.astype(f32), gamma.astype(f32), dt]),
                          dt * me_row[0]])

    Bp = 128 * pl.cdiv(B, 128)
    y0 = jnp.transpose(y.astype(f32), (2, 1, 0))
    if Bp != B:
        y0 = jnp.pad(y0, ((0, 0), (0, 0), (0, Bp - B)))

    n_tiles = pl.cdiv(Bp, LANES)
    blk_bytes = 4 * LANES * (3 + 6 * T)
    sol3, dif3 = pl.pallas_call(
        functools.partial(_sir_kernel, T=T),
        out_shape=[jax.ShapeDtypeStruct((3, T, Bp), f32)] * 2,
        grid=(n_tiles,),
        in_specs=[
            pl.BlockSpec(memory_space=pltpu.MemorySpace.SMEM),
            pl.BlockSpec((3, 1, LANES), lambda i: (0, 0, i)),
        ],
        out_specs=[pl.BlockSpec((3, T, LANES), lambda i: (0, 0, i))] * 2,
        compiler_params=pltpu.CompilerParams(
            dimension_semantics=("parallel",),
            vmem_limit_bytes=int(min(max(3 * blk_bytes, 16 << 20), 60 << 20))),
    )(sc, y0)

    solution = jnp.transpose(sol3[:, :, :B], (1, 2, 0))
    diff = jnp.transpose(dif3[:, :, :B], (1, 2, 0))
    return solution, diff, me


# speedup vs baseline: 1.0469x; 1.0022x over previous
import functools

import jax
import jax.numpy as jnp
from jax.experimental import pallas as pl
from jax.experimental.pallas import tpu as pltpu

HID = 20
HPAD = 128
LANES = 32768


def _me_kernel(t_ref, w1_ref, b1_ref, wh_ref, bh_ref, w4_ref, b4_ref, me_ref):
    h = jnp.tanh(t_ref[...] * w1_ref[...] + b1_ref[...])
    for i in range(2):
        h = jnp.tanh(jnp.dot(h, wh_ref[i],
                             preferred_element_type=jnp.float32) + bh_ref[i])
    z = jnp.dot(h, w4_ref[...], preferred_element_type=jnp.float32) + b4_ref[...]
    me_ref[...] = 0.5 * (jnp.tanh(0.5 * z) + 1.0)


def _sir_kernel(sc_ref,
                y0_ref,
                sol_ref, diff_ref,
                *, T):
    beta = sc_ref[0]
    gamma = sc_ref[1]
    dt = sc_ref[2]
    S = y0_ref[0]
    I = y0_ref[1]
    R = y0_ref[2]
    zero = jnp.zeros_like(S)
    I_hist = []
    for u in range(T):
        sol_ref[0, u:u + 1, :] = S
        sol_ref[1, u:u + 1, :] = I
        sol_ref[2, u:u + 1, :] = R
        if u == T - 1:
            diff_ref[0, u:u + 1, :] = zero
            diff_ref[1, u:u + 1, :] = zero
            diff_ref[2, u:u + 1, :] = zero
            break
        I_hist.append(I)
        acc = I_hist[0] * sc_ref[3 + T - 1 - u]
        for k in range(1, u + 1):
            acc = acc + I_hist[k] * sc_ref[3 + T - 1 - u + k]
        SI = S * I
        gI = gamma * I
        dSdt = acc - beta * SI
        dIdt = beta * SI - gI
        dRdt = gI - acc
        diff_ref[0, u:u + 1, :] = dSdt
        diff_ref[1, u:u + 1, :] = dIdt
        diff_ref[2, u:u + 1, :] = dRdt
        S = S + dt * dSdt
        I = I + dt * dIdt
        R = R + dt * dRdt


@jax.jit
def kernel(t, y, beta, gamma, w1, b1, w2, b2, w3, b3, w4, b4):
    T = t.shape[0]
    B = y.shape[0]
    f32 = jnp.float32
    t32 = t.astype(f32)
    dt = t32[0, 0] - t32[1, 0]

    w1p = jnp.pad(w1, ((0, 0), (0, HPAD - HID)))
    b1p = jnp.pad(b1, ((0, 0), (0, HPAD - HID)))
    whp = jnp.stack([jnp.pad(w2, ((0, HPAD - HID), (0, HPAD - HID))),
                     jnp.pad(w3, ((0, HPAD - HID), (0, HPAD - HID)))])
    bhp = jnp.stack([jnp.pad(b2, ((0, 0), (0, HPAD - HID))),
                     jnp.pad(b3, ((0, 0), (0, HPAD - HID)))])
    w4p = jnp.pad(w4, ((0, HPAD - HID), (0, 0)))
    b4p = b4.reshape(1, 1)

    vmem = pl.BlockSpec(memory_space=pltpu.MemorySpace.VMEM)
    me = pl.pallas_call(
        _me_kernel,
        out_shape=jax.ShapeDtypeStruct((T, 1), f32),
        in_specs=[vmem] * 7,
        out_specs=vmem,
    )(t32, w1p, b1p, whp, bhp, w4p, b4p)

    sc = jnp.concatenate([jnp.stack([beta.astype(f32), gamma.astype(f32), dt]),
                          dt * me[:, 0]])

    Bp = 128 * pl.cdiv(B, 128)
    y0 = jnp.transpose(y.astype(f32), (2, 1, 0))
    if Bp != B:
        y0 = jnp.pad(y0, ((0, 0), (0, 0), (0, Bp - B)))

    n_tiles = pl.cdiv(Bp, LANES)
    blk_bytes = 4 * LANES * (3 + 6 * T)
    sol3, dif3 = pl.pallas_call(
        functools.partial(_sir_kernel, T=T),
        out_shape=[jax.ShapeDtypeStruct((3, T, Bp), f32)] * 2,
        grid=(n_tiles,),
        in_specs=[
            pl.BlockSpec(memory_space=pltpu.MemorySpace.SMEM),
            pl.BlockSpec((3, 1, LANES), lambda i: (0, 0, i)),
        ],
        out_specs=[pl.BlockSpec((3, T, LANES), lambda i: (0, 0, i))] * 2,
        compiler_params=pltpu.CompilerParams(
            dimension_semantics=("parallel",),
            vmem_limit_bytes=int(min(max(3 * blk_bytes, 16 << 20), 60 << 20))),
    )(sc, y0)

    solution = jnp.transpose(sol3[:, :, :B], (1, 2, 0))
    diff = jnp.transpose(dif3[:, :, :B], (1, 2, 0))
    return solution, diff, me
